# baseline (speedup 1.0000x reference)
import jax
import jax.numpy as jnp
from jax.experimental import pallas as pl
from jax.experimental.pallas import tpu as pltpu

_W = 10000
_D = 2048
_B = 8192

_BUF_ROWS = 400
_X_ROWS = 512
_GRID = 25


def _body(pos_ref, x_ref, buf_ref, out_ref):
    i = pl.program_id(0)

    @pl.when(i == 0)
    def _init():
        out_ref[...] = jnp.zeros_like(out_ref)

    @pl.when(i < 25)
    def _buf_part():
        out_ref[...] += jnp.sum(buf_ref[...], axis=0, keepdims=True)

    @pl.when(i < 16)
    def _x_part():
        out_ref[...] += jnp.sum(x_ref[...], axis=0, keepdims=True) * (1.0 / _B)

    pos = pos_ref[0]

    @pl.when(i == pos // _BUF_ROWS)
    def _subtract_old_row():
        out_ref[...] -= buf_ref[pl.ds(pos % _BUF_ROWS, 1), :]

    @pl.when(i == _GRID - 1)
    def _finish():
        out_ref[...] *= (1.0 / _W)


def kernel(x, buf, pos):
    pos_arr = jnp.asarray(pos, dtype=jnp.int32).reshape((1,))
    out = pl.pallas_call(
        _body,
        grid=(_GRID,),
        in_specs=[
            pl.BlockSpec(memory_space=pltpu.SMEM),
            pl.BlockSpec((_X_ROWS, _D), lambda i: (jnp.minimum(i, 15), 0)),
            pl.BlockSpec((_BUF_ROWS, _D), lambda i: (i, 0)),
        ],
        out_specs=pl.BlockSpec((1, _D), lambda i: (0, 0)),
        out_shape=jax.ShapeDtypeStruct((1, _D), jnp.float32),
        compiler_params=pltpu.CompilerParams(
            dimension_semantics=("arbitrary",),
        ),
    )(pos_arr, x, buf)
    return out.reshape((_D,))


# speedup vs baseline: 1.0333x; 1.0333x over previous
import jax
import jax.numpy as jnp
from jax.experimental import pallas as pl
from jax.experimental.pallas import tpu as pltpu

_W = 10000
_D = 2048
_B = 8192

_BUF_ROWS = 1000
_X_ROWS = 2048
_GRID = 10


def _body(pos_ref, x_ref, buf_ref, out_ref):
    i = pl.program_id(0)

    @pl.when(i == 0)
    def _init():
        out_ref[...] = jnp.zeros_like(out_ref)

    @pl.when(i < 10)
    def _buf_part():
        out_ref[...] += jnp.sum(buf_ref[...], axis=0, keepdims=True)

    @pl.when(i < 4)
    def _x_part():
        out_ref[...] += jnp.sum(x_ref[...], axis=0, keepdims=True) * (1.0 / _B)

    pos = pos_ref[0]

    @pl.when(i == pos // _BUF_ROWS)
    def _subtract_old_row():
        out_ref[...] -= buf_ref[pl.ds(pos % _BUF_ROWS, 1), :]

    @pl.when(i == _GRID - 1)
    def _finish():
        out_ref[...] *= (1.0 / _W)


def kernel(x, buf, pos):
    pos_arr = jnp.asarray(pos, dtype=jnp.int32).reshape((1,))
    out = pl.pallas_call(
        _body,
        grid=(_GRID,),
        in_specs=[
            pl.BlockSpec(memory_space=pltpu.SMEM),
            pl.BlockSpec((_X_ROWS, _D), lambda i: (jnp.minimum(i, 3), 0)),
            pl.BlockSpec((_BUF_ROWS, _D), lambda i: (i, 0)),
        ],
        out_specs=pl.BlockSpec((1, _D), lambda i: (0, 0)),
        out_shape=jax.ShapeDtypeStruct((1, _D), jnp.float32),
        compiler_params=pltpu.CompilerParams(
            dimension_semantics=("arbitrary",),
        ),
    )(pos_arr, x, buf)
    return out.reshape((_D,))
